# baseline (speedup 1.0000x reference)
import functools

import jax
import jax.numpy as jnp
from jax import lax
from jax.experimental import pallas as pl
from jax.experimental.pallas import tpu as pltpu
from jax.experimental.pallas import tpu_sc as plsc

B = 16384
D = 32
PK = 128 // D
NC = 2
NS = 16
NW = NC * NS
B_PER_W = B // NW
CH = 64
NCH = B_PER_W // CH


def _sc_gather_body(uidx, midx, utab, mtab, uout, mout, idx_v, rows_v, sem):
    wid = lax.axis_index("s") * NC + lax.axis_index("c")
    base = wid * B_PER_W
    for t, idx_hbm in enumerate((uidx, midx)):
        for j in range(NCH):
            pltpu.sync_copy(idx_hbm.at[pl.ds(base + j * CH, CH)], idx_v.at[t * NCH + j])
    for r in range(2 * NCH):
        for k in range(CH // 16):
            s = pl.ds(k * 16, 16)
            idx_v[r, s] = idx_v[r, s] >> 2
    for t, (tab_hbm, out_hbm) in enumerate(((utab, uout), (mtab, mout))):
        copies = []
        for j in range(NCH):
            copies.append(
                pltpu.async_copy(tab_hbm.at[idx_v.at[t * NCH + j]], rows_v.at[j], sem)
            )
        for c in copies:
            c.wait()
        for j in range(NCH):
            pltpu.sync_copy(rows_v.at[j], out_hbm.at[pl.ds(base + j * CH, CH)])


def _sc_gather(user_inp, movie_inp, utab4, mtab4):
    mesh = plsc.VectorSubcoreMesh(
        core_axis_name="c", subcore_axis_name="s", num_cores=NC, num_subcores=NS
    )
    return pl.kernel(
        _sc_gather_body,
        out_type=(
            jax.ShapeDtypeStruct((B, 128), jnp.float32),
            jax.ShapeDtypeStruct((B, 128), jnp.float32),
        ),
        mesh=mesh,
        scratch_types=[
            pltpu.VMEM((2 * NCH, CH), jnp.int32),
            pltpu.VMEM((NCH, CH, 128), jnp.float32),
            pltpu.SemaphoreType.DMA,
        ],
    )(user_inp, movie_inp, utab4, mtab4)


def _tc_dense_body(ug, mg, uin, min_, oth, w_h, b_h, w_o, b_o, out):
    r_u = jnp.bitwise_and(uin[...], PK - 1)
    r_m = jnp.bitwise_and(min_[...], PK - 1)
    nb = ug.shape[0]
    ue = jnp.zeros((nb, D), jnp.float32)
    me = jnp.zeros((nb, D), jnp.float32)
    for r in range(PK):
        ue = ue + jnp.where(r_u == r, ug[:, r * D:(r + 1) * D], 0.0)
        me = me + jnp.where(r_m == r, mg[:, r * D:(r + 1) * D], 0.0)
    z = jnp.dot(oth[...], w_h[...], preferred_element_type=jnp.float32) + b_h[...]
    a = jnp.where(z >= 0, z, 0.01 * z)
    res = (
        jnp.dot(ue, w_o[0:D, :], preferred_element_type=jnp.float32)
        + jnp.dot(me, w_o[D:2 * D, :], preferred_element_type=jnp.float32)
        + jnp.dot(a, w_o[2 * D:, :], preferred_element_type=jnp.float32)
        + b_o[...]
    )
    out[...] = res[:, 0]


def kernel(user_inp, movie_inp, others_inp, user_table, movie_table, W_h, b_h, W_o, b_o):
    uin = user_inp.astype(jnp.int32)
    min_ = movie_inp.astype(jnp.int32)
    ug, mg = _sc_gather(
        uin, min_, user_table.reshape(-1, 128), movie_table.reshape(-1, 128)
    )
    blk = 2048
    out = pl.pallas_call(
        _tc_dense_body,
        grid=(B // blk,),
        in_specs=[
            pl.BlockSpec((blk, 128), lambda i: (i, 0)),
            pl.BlockSpec((blk, 128), lambda i: (i, 0)),
            pl.BlockSpec((blk, 1), lambda i: (i, 0)),
            pl.BlockSpec((blk, 1), lambda i: (i, 0)),
            pl.BlockSpec((blk, 64), lambda i: (i, 0)),
            pl.BlockSpec((64, 64), lambda i: (0, 0)),
            pl.BlockSpec((64,), lambda i: (0,)),
            pl.BlockSpec((128, 1), lambda i: (0, 0)),
            pl.BlockSpec((1,), lambda i: (0,)),
        ],
        out_specs=pl.BlockSpec((blk,), lambda i: (i,)),
        out_shape=jax.ShapeDtypeStruct((B,), jnp.float32),
    )(ug, mg, uin.reshape(B, 1), min_.reshape(B, 1), others_inp, W_h, b_h, W_o, b_o)
    return out


# speedup vs baseline: 2.8705x; 2.8705x over previous
import jax
import jax.numpy as jnp
from jax import lax
from jax.experimental import pallas as pl
from jax.experimental.pallas import tpu as pltpu
from jax.experimental.pallas import tpu_sc as plsc

B = 16384
D = 32
NC = 2
NS = 16
NW = NC * NS
B_PER_W = B // NW
GCH = 128
NGC = B_PER_W // GCH

BLKN = 4096


def _matvec_body(tT, w, out):
    out[...] = jnp.sum(tT[...] * w[...], axis=0)


def _score(table, w):
    n = table.shape[0]
    grid = (n + BLKN - 1) // BLKN
    return pl.pallas_call(
        _matvec_body,
        grid=(grid,),
        in_specs=[
            pl.BlockSpec((D, BLKN), lambda i: (0, i)),
            pl.BlockSpec((D, 1), lambda i: (0, 0)),
        ],
        out_specs=pl.BlockSpec((BLKN,), lambda i: (i,)),
        out_shape=jax.ShapeDtypeStruct((grid * BLKN,), jnp.float32),
    )(table.T, w)


def _sc_gather_body(uidx, midx, su, sm, gu, gm, idx_v, val_v, sem):
    wid = lax.axis_index("s") * NC + lax.axis_index("c")
    base = wid * B_PER_W
    for t, idx_hbm in enumerate((uidx, midx)):
        for j in range(NGC):
            pltpu.sync_copy(
                idx_hbm.at[pl.ds(base + j * GCH, GCH)], idx_v.at[t * NGC + j]
            )
    copies = []
    for t, s_hbm in enumerate((su, sm)):
        for j in range(NGC):
            r = t * NGC + j
            copies.append(pltpu.async_copy(s_hbm.at[idx_v.at[r]], val_v.at[r], sem))
    for c in copies:
        c.wait()
    for t, g_hbm in enumerate((gu, gm)):
        for j in range(NGC):
            pltpu.sync_copy(
                val_v.at[t * NGC + j], g_hbm.at[pl.ds(base + j * GCH, GCH)]
            )


def _sc_gather(uin, min_, su, sm):
    mesh = plsc.VectorSubcoreMesh(
        core_axis_name="c", subcore_axis_name="s", num_cores=NC, num_subcores=NS
    )
    return pl.kernel(
        _sc_gather_body,
        out_type=(
            jax.ShapeDtypeStruct((B,), jnp.float32),
            jax.ShapeDtypeStruct((B,), jnp.float32),
        ),
        mesh=mesh,
        scratch_types=[
            pltpu.VMEM((2 * NGC, GCH), jnp.int32),
            pltpu.VMEM((2 * NGC, GCH), jnp.float32),
            pltpu.SemaphoreType.DMA,
        ],
    )(uin, min_, su, sm)


def _tc_final_body(gu, gm, oth, w_h, b_h, w_o, b_o, out):
    z = jnp.dot(oth[...], w_h[...], preferred_element_type=jnp.float32) + b_h[...]
    a = jnp.where(z >= 0, z, 0.01 * z)
    d = jnp.dot(a, w_o[2 * D:, :], preferred_element_type=jnp.float32)
    out[...] = gu[...] + gm[...] + d[:, 0] + b_o[...]


def kernel(user_inp, movie_inp, others_inp, user_table, movie_table, W_h, b_h, W_o, b_o):
    uin = user_inp.astype(jnp.int32)
    min_ = movie_inp.astype(jnp.int32)
    sm = _score(movie_table, W_o[D:2 * D, :])
    su = _score(user_table, W_o[0:D, :])
    gu, gm = _sc_gather(uin, min_, su, sm)
    out = pl.pallas_call(
        _tc_final_body,
        out_shape=jax.ShapeDtypeStruct((B,), jnp.float32),
    )(gu, gm, others_inp, W_h, b_h, W_o, b_o)
    return out


# speedup vs baseline: 5.3045x; 1.8479x over previous
import jax
import jax.numpy as jnp
from jax import lax
from jax.experimental import pallas as pl
from jax.experimental.pallas import tpu as pltpu
from jax.experimental.pallas import tpu_sc as plsc

B = 16384
D = 32
NC = 2
NS = 16
NW = NC * NS
B_PER_W = B // NW
GCH = 128
NGC = B_PER_W // GCH

BLKN = 16384


def _matvec_body(tT, w, out):
    out[...] = jnp.sum(tT[...] * w[...], axis=0)


def _score(table, w):
    n = table.shape[0]
    grid = (n + BLKN - 1) // BLKN
    return pl.pallas_call(
        _matvec_body,
        grid=(grid,),
        in_specs=[
            pl.BlockSpec((D, BLKN), lambda i: (0, i)),
            pl.BlockSpec((D, 1), lambda i: (0, 0)),
        ],
        out_specs=pl.BlockSpec((BLKN,), lambda i: (i,)),
        out_shape=jax.ShapeDtypeStruct((grid * BLKN,), jnp.float32),
    )(table.T, w)


def _sc_gather_body(uidx, midx, su, sm, gu, gm, idx_v, val_v, sem):
    wid = lax.axis_index("s") * NC + lax.axis_index("c")
    base = wid * B_PER_W
    for t, idx_hbm in enumerate((uidx, midx)):
        for j in range(NGC):
            pltpu.sync_copy(
                idx_hbm.at[pl.ds(base + j * GCH, GCH)], idx_v.at[t * NGC + j]
            )
    copies = []
    for t, s_hbm in enumerate((su, sm)):
        for j in range(NGC):
            r = t * NGC + j
            copies.append(pltpu.async_copy(s_hbm.at[idx_v.at[r]], val_v.at[r], sem))
    for c in copies:
        c.wait()
    for t, g_hbm in enumerate((gu, gm)):
        for j in range(NGC):
            pltpu.sync_copy(
                val_v.at[t * NGC + j], g_hbm.at[pl.ds(base + j * GCH, GCH)]
            )


def _sc_gather(uin, min_, su, sm):
    mesh = plsc.VectorSubcoreMesh(
        core_axis_name="c", subcore_axis_name="s", num_cores=NC, num_subcores=NS
    )
    return pl.kernel(
        _sc_gather_body,
        out_type=(
            jax.ShapeDtypeStruct((B,), jnp.float32),
            jax.ShapeDtypeStruct((B,), jnp.float32),
        ),
        mesh=mesh,
        scratch_types=[
            pltpu.VMEM((2 * NGC, GCH), jnp.int32),
            pltpu.VMEM((2 * NGC, GCH), jnp.float32),
            pltpu.SemaphoreType.DMA,
        ],
    )(uin, min_, su, sm)


def _tc_final_body(gu, gm, oth, w_h, b_h, w_o, b_o, out):
    z = jnp.dot(oth[...], w_h[...], preferred_element_type=jnp.float32) + b_h[...]
    a = jnp.where(z >= 0, z, 0.01 * z)
    d = jnp.dot(a, w_o[2 * D:, :], preferred_element_type=jnp.float32)
    out[...] = gu[...] + gm[...] + d[:, 0] + b_o[...]


def kernel(user_inp, movie_inp, others_inp, user_table, movie_table, W_h, b_h, W_o, b_o):
    uin = user_inp.astype(jnp.int32)
    min_ = movie_inp.astype(jnp.int32)
    sm = _score(movie_table, W_o[D:2 * D, :])
    su = _score(user_table, W_o[0:D, :])
    gu, gm = _sc_gather(uin, min_, su, sm)
    out = pl.pallas_call(
        _tc_final_body,
        out_shape=jax.ShapeDtypeStruct((B,), jnp.float32),
    )(gu, gm, others_inp, W_h, b_h, W_o, b_o)
    return out


# speedup vs baseline: 6.2931x; 1.1864x over previous
import jax
import jax.numpy as jnp
from jax import lax
from jax.experimental import pallas as pl
from jax.experimental.pallas import tpu as pltpu
from jax.experimental.pallas import tpu_sc as plsc

B = 16384
D = 32
NC = 2
NS = 16
NW = NC * NS
B_PER_W = B // NW
GCH = 128
NGC = B_PER_W // GCH

BLKN = 32768


def _matvec_body(tT, w, out):
    out[...] = jnp.sum(tT[...] * w[...], axis=0)


def _score(table, w):
    n = table.shape[0]
    grid = (n + BLKN - 1) // BLKN
    return pl.pallas_call(
        _matvec_body,
        grid=(grid,),
        in_specs=[
            pl.BlockSpec((D, BLKN), lambda i: (0, i)),
            pl.BlockSpec((D, 1), lambda i: (0, 0)),
        ],
        out_specs=pl.BlockSpec((BLKN,), lambda i: (i,)),
        out_shape=jax.ShapeDtypeStruct((grid * BLKN,), jnp.float32),
    )(table.T, w)


def _sc_gather_body(uidx, midx, su, sm, gu, gm, idx_v, val_v, sem):
    wid = lax.axis_index("s") * NC + lax.axis_index("c")
    base = wid * B_PER_W
    for t, idx_hbm in enumerate((uidx, midx)):
        for j in range(NGC):
            pltpu.sync_copy(
                idx_hbm.at[pl.ds(base + j * GCH, GCH)], idx_v.at[t * NGC + j]
            )
    copies = []
    for t, s_hbm in enumerate((su, sm)):
        for j in range(NGC):
            r = t * NGC + j
            copies.append(pltpu.async_copy(s_hbm.at[idx_v.at[r]], val_v.at[r], sem))
    for c in copies:
        c.wait()
    for t, g_hbm in enumerate((gu, gm)):
        for j in range(NGC):
            pltpu.sync_copy(
                val_v.at[t * NGC + j], g_hbm.at[pl.ds(base + j * GCH, GCH)]
            )


def _sc_gather(uin, min_, su, sm):
    mesh = plsc.VectorSubcoreMesh(
        core_axis_name="c", subcore_axis_name="s", num_cores=NC, num_subcores=NS
    )
    return pl.kernel(
        _sc_gather_body,
        out_type=(
            jax.ShapeDtypeStruct((B,), jnp.float32),
            jax.ShapeDtypeStruct((B,), jnp.float32),
        ),
        mesh=mesh,
        scratch_types=[
            pltpu.VMEM((2 * NGC, GCH), jnp.int32),
            pltpu.VMEM((2 * NGC, GCH), jnp.float32),
            pltpu.SemaphoreType.DMA,
        ],
    )(uin, min_, su, sm)


def _tc_final_body(gu, gm, oth, w_h, b_h, w_o, b_o, out):
    z = jnp.dot(oth[...], w_h[...], preferred_element_type=jnp.float32) + b_h[...]
    a = jnp.where(z >= 0, z, 0.01 * z)
    d = jnp.dot(a, w_o[2 * D:, :], preferred_element_type=jnp.float32)
    out[...] = gu[...] + gm[...] + d[:, 0] + b_o[...]


def kernel(user_inp, movie_inp, others_inp, user_table, movie_table, W_h, b_h, W_o, b_o):
    uin = user_inp.astype(jnp.int32)
    min_ = movie_inp.astype(jnp.int32)
    sm = _score(movie_table, W_o[D:2 * D, :])
    su = _score(user_table, W_o[0:D, :])
    gu, gm = _sc_gather(uin, min_, su, sm)
    out = pl.pallas_call(
        _tc_final_body,
        out_shape=jax.ShapeDtypeStruct((B,), jnp.float32),
    )(gu, gm, others_inp, W_h, b_h, W_o, b_o)
    return out


# speedup vs baseline: 6.8138x; 1.0827x over previous
import jax
import jax.numpy as jnp
from jax import lax
from jax.experimental import pallas as pl
from jax.experimental.pallas import tpu as pltpu
from jax.experimental.pallas import tpu_sc as plsc

B = 16384
D = 32
NC = 2
NS = 16
NW = NC * NS
B_PER_W = B // NW
GCH = 128
NGC = B_PER_W // GCH

BLKN = 65536


def _matvec_body(tT, w, out):
    out[...] = jnp.sum(tT[...] * w[...], axis=0)


def _score(table, w):
    n = table.shape[0]
    grid = (n + BLKN - 1) // BLKN
    return pl.pallas_call(
        _matvec_body,
        grid=(grid,),
        in_specs=[
            pl.BlockSpec((D, BLKN), lambda i: (0, i)),
            pl.BlockSpec((D, 1), lambda i: (0, 0)),
        ],
        out_specs=pl.BlockSpec((BLKN,), lambda i: (i,)),
        out_shape=jax.ShapeDtypeStruct((grid * BLKN,), jnp.float32),
    )(table.T, w)


def _sc_gather_body(uidx, midx, su, sm, gu, gm, idx_v, val_v, sem):
    wid = lax.axis_index("s") * NC + lax.axis_index("c")
    base = wid * B_PER_W
    for t, idx_hbm in enumerate((uidx, midx)):
        for j in range(NGC):
            pltpu.sync_copy(
                idx_hbm.at[pl.ds(base + j * GCH, GCH)], idx_v.at[t * NGC + j]
            )
    copies = []
    for t, s_hbm in enumerate((su, sm)):
        for j in range(NGC):
            r = t * NGC + j
            copies.append(pltpu.async_copy(s_hbm.at[idx_v.at[r]], val_v.at[r], sem))
    for c in copies:
        c.wait()
    for t, g_hbm in enumerate((gu, gm)):
        for j in range(NGC):
            pltpu.sync_copy(
                val_v.at[t * NGC + j], g_hbm.at[pl.ds(base + j * GCH, GCH)]
            )


def _sc_gather(uin, min_, su, sm):
    mesh = plsc.VectorSubcoreMesh(
        core_axis_name="c", subcore_axis_name="s", num_cores=NC, num_subcores=NS
    )
    return pl.kernel(
        _sc_gather_body,
        out_type=(
            jax.ShapeDtypeStruct((B,), jnp.float32),
            jax.ShapeDtypeStruct((B,), jnp.float32),
        ),
        mesh=mesh,
        scratch_types=[
            pltpu.VMEM((2 * NGC, GCH), jnp.int32),
            pltpu.VMEM((2 * NGC, GCH), jnp.float32),
            pltpu.SemaphoreType.DMA,
        ],
    )(uin, min_, su, sm)


def _tc_final_body(gu, gm, oth, w_h, b_h, w_o, b_o, out):
    z = jnp.dot(oth[...], w_h[...], preferred_element_type=jnp.float32) + b_h[...]
    a = jnp.where(z >= 0, z, 0.01 * z)
    d = jnp.dot(a, w_o[2 * D:, :], preferred_element_type=jnp.float32)
    out[...] = gu[...] + gm[...] + d[:, 0] + b_o[...]


def kernel(user_inp, movie_inp, others_inp, user_table, movie_table, W_h, b_h, W_o, b_o):
    uin = user_inp.astype(jnp.int32)
    min_ = movie_inp.astype(jnp.int32)
    sm = _score(movie_table, W_o[D:2 * D, :])
    su = _score(user_table, W_o[0:D, :])
    gu, gm = _sc_gather(uin, min_, su, sm)
    out = pl.pallas_call(
        _tc_final_body,
        out_shape=jax.ShapeDtypeStruct((B,), jnp.float32),
    )(gu, gm, others_inp, W_h, b_h, W_o, b_o)
    return out
